# Initial kernel scaffold; baseline (speedup 1.0000x reference)
import jax
import jax.numpy as jnp
from jax.experimental import pallas as pl


def kernel(x, edge_index, batch, W_emb, b_emb, W_root, b_root, W_conv, W1, b1, W2, b2):
    raise NotImplementedError("write your pallas kernel here")


# speedup vs baseline: 4.4331x; 4.4331x over previous
import jax
import jax.numpy as jnp
from jax import lax
from jax.experimental import pallas as pl
from jax.experimental.pallas import tpu as pltpu
from jax.experimental.pallas import tpu_sc as plsc

N, D, L, G, E, NLAYERS = 50000, 128, 2, 64, 400000, 4

NB = 4
BUCKET = 12544
NP = NB * BUCKET
ACC_ROWS = 12560
NT = 32
SLICE = 12544
EPAD = NT * SLICE
CAP = SLICE + 128
W = 128
NWIN = 8
WIN = SLICE // NWIN
RPT = BUCKET // 16

R_BLK = 784
N_BLKS = NP // R_BLK

_mesh = plsc.VectorSubcoreMesh(core_axis_name="c", subcore_axis_name="s")
_sc_params = pltpu.CompilerParams(needs_layout_passes=False)


def _bucketize_body(src2d, dst2d, srcb, dstb, counts, sb0, sb1, sb2, sb3,
                    db0, db1, db2, db3, swin, dwin, cvec):
    sbufs = (sb0, sb1, sb2, sb3)
    dbufs = (db0, db1, db2, db3)
    c = lax.axis_index("c")
    s = lax.axis_index("s")
    p = c * 16 + s
    lanes = jnp.arange(16, dtype=jnp.int32)
    pad_src = lanes
    pad_dst = BUCKET + (lanes & 7)
    for rel in range(L):
        cnts = (jnp.int32(0),) * NB
        for w in range(NWIN):
            off = rel * EPAD + p * SLICE + w * WIN
            pltpu.sync_copy(src2d.at[pl.ds(off, WIN)], swin)
            pltpu.sync_copy(dst2d.at[pl.ds(off, WIN)], dwin)

            def body(k, cnts):
                sv = swin[pl.ds(k * 16, 16)]
                dv = dwin[pl.ds(k * 16, 16)]
                bkt = ((dv >= BUCKET).astype(jnp.int32)
                       + (dv >= 2 * BUCKET).astype(jnp.int32)
                       + (dv >= 3 * BUCKET).astype(jnp.int32))
                valid = dv >= 0
                dloc = dv - bkt * BUCKET
                out = []
                for b in range(NB):
                    msk = jnp.logical_and(bkt == b, valid)
                    plsc.store_compressed(
                        sbufs[b].at[pl.ds(cnts[b], 16)], sv, mask=msk)
                    plsc.store_compressed(
                        dbufs[b].at[pl.ds(cnts[b], 16)], dloc, mask=msk)
                    npop = jnp.max(plsc.all_reduce_population_count(msk))
                    out.append(cnts[b] + npop)
                return tuple(out)

            cnts = lax.fori_loop(0, WIN // 16, body, cnts)
        cw = jnp.zeros((16,), jnp.int32)
        for b in range(NB):
            cb = cnts[b]
            for j in range(8):
                sbufs[b][pl.ds(cb + j * 16, 16)] = pad_src
                dbufs[b][pl.ds(cb + j * 16, 16)] = pad_dst
            padded = ((cb + (W - 1)) // W) * W
            cw = jnp.where(lanes == b, padded, cw)
            boff = ((rel * NB + b) * NT + p) * CAP
            pltpu.sync_copy(sbufs[b], srcb.at[pl.ds(boff, CAP)])
            pltpu.sync_copy(dbufs[b], dstb.at[pl.ds(boff, CAP)])
        cvec[...] = cw
        pltpu.sync_copy(cvec, counts.at[pl.ds((rel * NT + p) * 16, 16)])


_bucketize = pl.kernel(
    _bucketize_body,
    out_type=(
        jax.ShapeDtypeStruct((L * NB * NT * CAP,), jnp.int32),
        jax.ShapeDtypeStruct((L * NB * NT * CAP,), jnp.int32),
        jax.ShapeDtypeStruct((L * NT * 16,), jnp.int32),
    ),
    mesh=_mesh,
    compiler_params=_sc_params,
    scratch_types=(
        [pltpu.VMEM((CAP,), jnp.int32)] * 8
        + [pltpu.VMEM((WIN,), jnp.int32),
           pltpu.VMEM((WIN,), jnp.int32),
           pltpu.VMEM((16,), jnp.int32)]
    ),
)


def _layer_sc_body(r_hbm, m0_hbm, m1_hbm, srcb, dstb, counts, out_hbm,
                   acc, sidx, didx, rows, cvec, gsem):
    c = lax.axis_index("c")
    s = lax.axis_index("s")
    lanes = jnp.arange(16, dtype=jnp.int32)
    for cpass in range(2):
        b = c * 2 + cpass
        pltpu.sync_copy(r_hbm.at[pl.ds(b * BUCKET + s * RPT, RPT)],
                        acc.at[pl.ds(s * RPT, RPT)])
        plsc.subcore_barrier()
        for rel in range(L):
            m_hbm = (m0_hbm, m1_hbm)[rel]
            for prod_off in (0, 16):
                prod = s + prod_off
                pltpu.sync_copy(counts.at[pl.ds((rel * NT + prod) * 16, 16)],
                                cvec)
                cnt = jnp.max(jnp.where(lanes == b, cvec[...], 0))
                base = ((rel * NB + b) * NT + prod) * CAP

                def wbody(w, carry):
                    pltpu.sync_copy(srcb.at[pl.ds(base + w * W, W)], sidx)
                    pltpu.sync_copy(dstb.at[pl.ds(base + w * W, W)], didx)
                    pltpu.async_copy(m_hbm.at[sidx], rows, gsem).wait()
                    pltpu.sync_copy(rows, acc.at[didx], add=True)
                    return carry

                lax.fori_loop(0, cnt // W, wbody, 0)
        plsc.subcore_barrier()
        pltpu.sync_copy(acc.at[pl.ds(s * RPT, RPT)],
                        out_hbm.at[pl.ds(b * BUCKET + s * RPT, RPT)])
        plsc.subcore_barrier()


_layer_sc = pl.kernel(
    _layer_sc_body,
    out_type=jax.ShapeDtypeStruct((NP, D), jnp.float32),
    mesh=_mesh,
    compiler_params=_sc_params,
    scratch_types=[
        pltpu.VMEM_SHARED((ACC_ROWS, D), jnp.float32),
        pltpu.VMEM((W,), jnp.int32),
        pltpu.VMEM((W,), jnp.int32),
        pltpu.VMEM((W, D), jnp.float32),
        pltpu.VMEM((16,), jnp.int32),
        pltpu.SemaphoreType.DMA,
    ],
)


def _mats_body(do_relu, h_ref, wrt_ref, br_ref, wc0t_ref, wc1t_ref,
               wet_ref, be_ref, r_ref, m0_ref, m1_ref):
    h = h_ref[...]
    if do_relu:
        h = jnp.maximum(h, 0.0)
    if wet_ref is not None:
        h = (jnp.dot(h, wet_ref[...], preferred_element_type=jnp.float32)
             + be_ref[...])
    r_ref[...] = (jnp.dot(h, wrt_ref[...], preferred_element_type=jnp.float32)
                  + br_ref[...])
    m0_ref[...] = jnp.dot(h, wc0t_ref[...], preferred_element_type=jnp.float32)
    m1_ref[...] = jnp.dot(h, wc1t_ref[...], preferred_element_type=jnp.float32)


def _make_mats(with_emb, do_relu):
    full = pl.BlockSpec((D, D), lambda i: (0, 0))
    bias = pl.BlockSpec((1, D), lambda i: (0, 0))
    row = pl.BlockSpec((R_BLK, D), lambda i: (i, 0))
    in_specs = [row, full, bias, full, full]
    if with_emb:
        in_specs += [full, bias]

    def body(*refs):
        if with_emb:
            _mats_body(do_relu, *refs[:7], *refs[7:])
        else:
            _mats_body(do_relu, *refs[:5], None, None, *refs[5:])

    return pl.pallas_call(
        body,
        grid=(N_BLKS,),
        in_specs=in_specs,
        out_specs=[row, row, row],
        out_shape=[jax.ShapeDtypeStruct((NP, D), jnp.float32)] * 3,
    )


_mats_emb = _make_mats(True, False)
_mats_relu = _make_mats(False, True)


def _pool_body(h_ref, batch_ref, w1t_ref, b1_ref, w2t_ref, b2_ref,
               out_ref, acc_ref):
    i = pl.program_id(0)

    @pl.when(i == 0)
    def _():
        acc_ref[...] = jnp.zeros_like(acc_ref)

    bvec = batch_ref[0, 0, :]
    onehot = (bvec[:, None]
              == lax.broadcasted_iota(jnp.int32, (R_BLK, G), 1)
              ).astype(jnp.float32)
    acc_ref[...] += lax.dot_general(
        onehot, h_ref[...], (((0,), (0,)), ((), ())),
        preferred_element_type=jnp.float32)

    @pl.when(i == N_BLKS - 1)
    def _():
        hid = jnp.maximum(
            jnp.dot(acc_ref[...], w1t_ref[...],
                    preferred_element_type=jnp.float32) + b1_ref[...], 0.0)
        o = jnp.dot(hid, w2t_ref[...],
                    preferred_element_type=jnp.float32) + b2_ref[...]
        out_ref[...] = o.reshape(1, G)


_pool = pl.pallas_call(
    _pool_body,
    grid=(N_BLKS,),
    in_specs=[
        pl.BlockSpec((R_BLK, D), lambda i: (i, 0)),
        pl.BlockSpec((1, 1, R_BLK), lambda i: (i, 0, 0)),
        pl.BlockSpec((D, D), lambda i: (0, 0)),
        pl.BlockSpec((1, D), lambda i: (0, 0)),
        pl.BlockSpec((D, 1), lambda i: (0, 0)),
        pl.BlockSpec((1, 1), lambda i: (0, 0)),
    ],
    out_specs=pl.BlockSpec((1, G), lambda i: (0, 0)),
    out_shape=jax.ShapeDtypeStruct((1, G), jnp.float32),
    scratch_shapes=[pltpu.VMEM((G, D), jnp.float32)],
)


def kernel(x, edge_index, batch, W_emb, b_emb, W_root, b_root, W_conv,
           W1, b1, W2, b2):
    src = edge_index[:, 0, :]
    dst = edge_index[:, 1, :]
    pad = EPAD - E
    src_flat = jnp.concatenate(
        [src, jnp.zeros((L, pad), jnp.int32)], axis=1).reshape(-1)
    dst_flat = jnp.concatenate(
        [dst, jnp.full((L, pad), -1, jnp.int32)], axis=1).reshape(-1)
    srcb, dstb, counts = _bucketize(src_flat, dst_flat)

    h = jnp.pad(x, ((0, NP - N), (0, 0)))
    for l in range(NLAYERS):
        mats = _mats_emb if l == 0 else _mats_relu
        args = [h, W_root[l].T, b_root[l].reshape(1, D),
                W_conv[l, 0].T, W_conv[l, 1].T]
        if l == 0:
            args += [W_emb.T, b_emb.reshape(1, D)]
        r, m0, m1 = mats(*args)
        h = _layer_sc(r, m0, m1, srcb, dstb, counts)

    batch_p = jnp.pad(batch, (0, NP - N), constant_values=-1)
    batch3d = batch_p.reshape(N_BLKS, 1, R_BLK)
    out = _pool(h, batch3d, W1.T, b1.reshape(1, D), W2.T, b2.reshape(1, 1))
    return out.reshape(G)


# speedup vs baseline: 6.0928x; 1.3744x over previous
import jax
import jax.numpy as jnp
from jax import lax
from jax.experimental import pallas as pl
from jax.experimental.pallas import tpu as pltpu
from jax.experimental.pallas import tpu_sc as plsc

N, D, L, G, E, NLAYERS = 50000, 128, 2, 64, 400000, 4

NB = 4
BUCKET = 12544
NP = NB * BUCKET
ACC_ROWS = 12560
NT = 32
SLICE = 12544
EPAD = NT * SLICE
W = 112
CAP = SLICE + 2 * W
NWIN = 8
WIN = SLICE // NWIN
RPT = BUCKET // 16

R_BLK = 784
N_BLKS = NP // R_BLK

_mesh = plsc.VectorSubcoreMesh(core_axis_name="c", subcore_axis_name="s")
_sc_params = pltpu.CompilerParams(needs_layout_passes=False)


def _bucketize_body(src2d, dst2d, srcb, dstb, counts, sb0, sb1, sb2, sb3,
                    db0, db1, db2, db3, swin, dwin, cvec):
    sbufs = (sb0, sb1, sb2, sb3)
    dbufs = (db0, db1, db2, db3)
    c = lax.axis_index("c")
    s = lax.axis_index("s")
    p = c * 16 + s
    lanes = jnp.arange(16, dtype=jnp.int32)
    pad_src = lanes
    pad_dst = BUCKET + (lanes & 7)
    for rel in range(L):
        cnts = (jnp.int32(0),) * NB
        for w in range(NWIN):
            off = rel * EPAD + p * SLICE + w * WIN
            pltpu.sync_copy(src2d.at[pl.ds(off, WIN)], swin)
            pltpu.sync_copy(dst2d.at[pl.ds(off, WIN)], dwin)

            def body(k, cnts):
                sv = swin[pl.ds(k * 16, 16)]
                dv = dwin[pl.ds(k * 16, 16)]
                bkt = ((dv >= BUCKET).astype(jnp.int32)
                       + (dv >= 2 * BUCKET).astype(jnp.int32)
                       + (dv >= 3 * BUCKET).astype(jnp.int32))
                valid = dv >= 0
                dloc = dv - bkt * BUCKET
                out = []
                for b in range(NB):
                    msk = jnp.logical_and(bkt == b, valid)
                    plsc.store_compressed(
                        sbufs[b].at[pl.ds(cnts[b], 16)], sv, mask=msk)
                    plsc.store_compressed(
                        dbufs[b].at[pl.ds(cnts[b], 16)], dloc, mask=msk)
                    npop = jnp.max(plsc.all_reduce_population_count(msk))
                    out.append(cnts[b] + npop)
                return tuple(out)

            cnts = lax.fori_loop(0, WIN // 16, body, cnts)
        cw = jnp.zeros((16,), jnp.int32)
        for b in range(NB):
            cb = cnts[b]
            for j in range(2 * W // 16):
                sbufs[b][pl.ds(cb + j * 16, 16)] = pad_src
                dbufs[b][pl.ds(cb + j * 16, 16)] = pad_dst
            padded = ((cb + (2 * W - 1)) // (2 * W)) * (2 * W)
            cw = jnp.where(lanes == b, padded, cw)
            boff = ((rel * NB + b) * NT + p) * CAP
            pltpu.sync_copy(sbufs[b], srcb.at[pl.ds(boff, CAP)])
            pltpu.sync_copy(dbufs[b], dstb.at[pl.ds(boff, CAP)])
        cvec[...] = cw
        pltpu.sync_copy(cvec, counts.at[pl.ds((rel * NT + p) * 16, 16)])


_bucketize = pl.kernel(
    _bucketize_body,
    out_type=(
        jax.ShapeDtypeStruct((L * NB * NT * CAP,), jnp.int32),
        jax.ShapeDtypeStruct((L * NB * NT * CAP,), jnp.int32),
        jax.ShapeDtypeStruct((L * NT * 16,), jnp.int32),
    ),
    mesh=_mesh,
    compiler_params=_sc_params,
    scratch_types=(
        [pltpu.VMEM((CAP,), jnp.int32)] * 8
        + [pltpu.VMEM((WIN,), jnp.int32),
           pltpu.VMEM((WIN,), jnp.int32),
           pltpu.VMEM((16,), jnp.int32)]
    ),
)


def _layer_sc_body(r_hbm, m0_hbm, m1_hbm, srcb, dstb, counts, out_hbm,
                   acc, idxa, idxb, rowsa, rowsb, cvec,
                   isema, isemb, gsema, gsemb):
    c = lax.axis_index("c")
    s = lax.axis_index("s")
    lanes = jnp.arange(16, dtype=jnp.int32)
    for cpass in range(2):
        b = c * 2 + cpass
        pltpu.sync_copy(r_hbm.at[pl.ds(b * BUCKET + s * RPT, RPT)],
                        acc.at[pl.ds(s * RPT, RPT)])
        plsc.subcore_barrier()
        for rel in range(L):
            m_hbm = (m0_hbm, m1_hbm)[rel]
            for prod_off in (0, 16):
                prod = s + prod_off
                pltpu.sync_copy(counts.at[pl.ds((rel * NT + prod) * 16, 16)],
                                cvec)
                cnt = jnp.max(jnp.where(lanes == b, cvec[...], 0))
                base = ((rel * NB + b) * NT + prod) * CAP
                nwin = cnt // W

                def start_idx(w, idx, isem):
                    pltpu.async_copy(
                        srcb.at[pl.ds(base + w * W, W)], idx.at[0], isem
                    ).wait()
                    pltpu.async_copy(
                        dstb.at[pl.ds(base + w * W, W)], idx.at[1], isem
                    ).wait()

                def gstart(idx, rows, gsem):
                    pltpu.async_copy(m_hbm.at[idx.at[0]], rows, gsem)

                def gwait(idx, rows, gsem):
                    pltpu.make_async_copy(
                        m_hbm.at[idx.at[0]], rows, gsem).wait()

                @pl.when(nwin > 0)
                def _():
                    start_idx(0, idxa, isema)
                    gstart(idxa, rowsa, gsema)

                    def body2(w2, carry):
                        w = 2 * w2
                        start_idx(w + 1, idxb, isemb)
                        gstart(idxb, rowsb, gsemb)
                        gwait(idxa, rowsa, gsema)
                        pltpu.sync_copy(rowsa, acc.at[idxa.at[1]], add=True)

                        @pl.when(w + 2 < nwin)
                        def _():
                            start_idx(w + 2, idxa, isema)
                            gstart(idxa, rowsa, gsema)

                        gwait(idxb, rowsb, gsemb)
                        pltpu.sync_copy(rowsb, acc.at[idxb.at[1]], add=True)
                        return carry

                    lax.fori_loop(0, nwin // 2, body2, 0)
        plsc.subcore_barrier()
        pltpu.sync_copy(acc.at[pl.ds(s * RPT, RPT)],
                        out_hbm.at[pl.ds(b * BUCKET + s * RPT, RPT)])
        plsc.subcore_barrier()


_layer_sc = pl.kernel(
    _layer_sc_body,
    out_type=jax.ShapeDtypeStruct((NP, D), jnp.float32),
    mesh=_mesh,
    compiler_params=_sc_params,
    scratch_types=[
        pltpu.VMEM_SHARED((ACC_ROWS, D), jnp.float32),
        pltpu.VMEM((2, W), jnp.int32),
        pltpu.VMEM((2, W), jnp.int32),
        pltpu.VMEM((W, D), jnp.float32),
        pltpu.VMEM((W, D), jnp.float32),
        pltpu.VMEM((16,), jnp.int32),
        pltpu.SemaphoreType.DMA,
        pltpu.SemaphoreType.DMA,
        pltpu.SemaphoreType.DMA,
        pltpu.SemaphoreType.DMA,
    ],
)


def _mats_body(do_relu, h_ref, wrt_ref, br_ref, wc0t_ref, wc1t_ref,
               wet_ref, be_ref, r_ref, m0_ref, m1_ref):
    h = h_ref[...]
    if do_relu:
        h = jnp.maximum(h, 0.0)
    if wet_ref is not None:
        h = (jnp.dot(h, wet_ref[...], preferred_element_type=jnp.float32)
             + be_ref[...])
    r_ref[...] = (jnp.dot(h, wrt_ref[...], preferred_element_type=jnp.float32)
                  + br_ref[...])
    m0_ref[...] = jnp.dot(h, wc0t_ref[...], preferred_element_type=jnp.float32)
    m1_ref[...] = jnp.dot(h, wc1t_ref[...], preferred_element_type=jnp.float32)


def _make_mats(with_emb, do_relu):
    full = pl.BlockSpec((D, D), lambda i: (0, 0))
    bias = pl.BlockSpec((1, D), lambda i: (0, 0))
    row = pl.BlockSpec((R_BLK, D), lambda i: (i, 0))
    in_specs = [row, full, bias, full, full]
    if with_emb:
        in_specs += [full, bias]

    def body(*refs):
        if with_emb:
            _mats_body(do_relu, *refs[:7], *refs[7:])
        else:
            _mats_body(do_relu, *refs[:5], None, None, *refs[5:])

    return pl.pallas_call(
        body,
        grid=(N_BLKS,),
        in_specs=in_specs,
        out_specs=[row, row, row],
        out_shape=[jax.ShapeDtypeStruct((NP, D), jnp.float32)] * 3,
    )


_mats_emb = _make_mats(True, False)
_mats_relu = _make_mats(False, True)


def _pool_body(h_ref, batch_ref, w1t_ref, b1_ref, w2t_ref, b2_ref,
               out_ref, acc_ref):
    i = pl.program_id(0)

    @pl.when(i == 0)
    def _():
        acc_ref[...] = jnp.zeros_like(acc_ref)

    bvec = batch_ref[0, 0, :]
    onehot = (bvec[:, None]
              == lax.broadcasted_iota(jnp.int32, (R_BLK, G), 1)
              ).astype(jnp.float32)
    acc_ref[...] += lax.dot_general(
        onehot, h_ref[...], (((0,), (0,)), ((), ())),
        preferred_element_type=jnp.float32)

    @pl.when(i == N_BLKS - 1)
    def _():
        hid = jnp.maximum(
            jnp.dot(acc_ref[...], w1t_ref[...],
                    preferred_element_type=jnp.float32) + b1_ref[...], 0.0)
        o = jnp.dot(hid, w2t_ref[...],
                    preferred_element_type=jnp.float32) + b2_ref[...]
        out_ref[...] = o.reshape(1, G)


_pool = pl.pallas_call(
    _pool_body,
    grid=(N_BLKS,),
    in_specs=[
        pl.BlockSpec((R_BLK, D), lambda i: (i, 0)),
        pl.BlockSpec((1, 1, R_BLK), lambda i: (i, 0, 0)),
        pl.BlockSpec((D, D), lambda i: (0, 0)),
        pl.BlockSpec((1, D), lambda i: (0, 0)),
        pl.BlockSpec((D, 1), lambda i: (0, 0)),
        pl.BlockSpec((1, 1), lambda i: (0, 0)),
    ],
    out_specs=pl.BlockSpec((1, G), lambda i: (0, 0)),
    out_shape=jax.ShapeDtypeStruct((1, G), jnp.float32),
    scratch_shapes=[pltpu.VMEM((G, D), jnp.float32)],
)


def kernel(x, edge_index, batch, W_emb, b_emb, W_root, b_root, W_conv,
           W1, b1, W2, b2):
    src = edge_index[:, 0, :]
    dst = edge_index[:, 1, :]
    pad = EPAD - E
    src_flat = jnp.concatenate(
        [src, jnp.zeros((L, pad), jnp.int32)], axis=1).reshape(-1)
    dst_flat = jnp.concatenate(
        [dst, jnp.full((L, pad), -1, jnp.int32)], axis=1).reshape(-1)
    srcb, dstb, counts = _bucketize(src_flat, dst_flat)

    h = jnp.pad(x, ((0, NP - N), (0, 0)))
    for l in range(NLAYERS):
        mats = _mats_emb if l == 0 else _mats_relu
        args = [h, W_root[l].T, b_root[l].reshape(1, D),
                W_conv[l, 0].T, W_conv[l, 1].T]
        if l == 0:
            args += [W_emb.T, b_emb.reshape(1, D)]
        r, m0, m1 = mats(*args)
        h = _layer_sc(r, m0, m1, srcb, dstb, counts)

    batch_p = jnp.pad(batch, (0, NP - N), constant_values=-1)
    batch3d = batch_p.reshape(N_BLKS, 1, R_BLK)
    out = _pool(h, batch3d, W1.T, b1.reshape(1, D), W2.T, b2.reshape(1, 1))
    return out.reshape(G)


# speedup vs baseline: 6.5744x; 1.0791x over previous
import jax
import jax.numpy as jnp
from jax import lax
from jax.experimental import pallas as pl
from jax.experimental.pallas import tpu as pltpu
from jax.experimental.pallas import tpu_sc as plsc

N, D, L, G, E, NLAYERS = 50000, 128, 2, 64, 400000, 4

NB = 4
BUCKET = 12544
NP = NB * BUCKET
ACC_ROWS = 12560
NT = 32
SLICE = 12544
EPAD = NT * SLICE
W = 112
CAP = SLICE + 4 * W
NWIN = 8
WIN = SLICE // NWIN
RPT = BUCKET // 16

R_BLK = 784
N_BLKS = NP // R_BLK

_mesh = plsc.VectorSubcoreMesh(core_axis_name="c", subcore_axis_name="s")
_sc_params = pltpu.CompilerParams(needs_layout_passes=False)


def _bucketize_body(src2d, dst2d, srcb, dstb, counts, sb0, sb1, sb2, sb3,
                    db0, db1, db2, db3, swin, dwin, cvec):
    sbufs = (sb0, sb1, sb2, sb3)
    dbufs = (db0, db1, db2, db3)
    c = lax.axis_index("c")
    s = lax.axis_index("s")
    p = c * 16 + s
    lanes = jnp.arange(16, dtype=jnp.int32)
    pad_src = lanes
    pad_dst = BUCKET + (lanes & 7)
    for rel in range(L):
        cnts = (jnp.int32(0),) * NB
        for w in range(NWIN):
            off = rel * EPAD + p * SLICE + w * WIN
            pltpu.sync_copy(src2d.at[pl.ds(off, WIN)], swin)
            pltpu.sync_copy(dst2d.at[pl.ds(off, WIN)], dwin)

            def body(k, cnts):
                sv = swin[pl.ds(k * 16, 16)]
                dv = dwin[pl.ds(k * 16, 16)]
                bkt = ((dv >= BUCKET).astype(jnp.int32)
                       + (dv >= 2 * BUCKET).astype(jnp.int32)
                       + (dv >= 3 * BUCKET).astype(jnp.int32))
                valid = dv >= 0
                dloc = dv - bkt * BUCKET
                out = []
                for b in range(NB):
                    msk = jnp.logical_and(bkt == b, valid)
                    plsc.store_compressed(
                        sbufs[b].at[pl.ds(cnts[b], 16)], sv, mask=msk)
                    plsc.store_compressed(
                        dbufs[b].at[pl.ds(cnts[b], 16)], dloc, mask=msk)
                    npop = jnp.max(plsc.all_reduce_population_count(msk))
                    out.append(cnts[b] + npop)
                return tuple(out)

            cnts = lax.fori_loop(0, WIN // 16, body, cnts)
        cw = jnp.zeros((16,), jnp.int32)
        for b in range(NB):
            cb = cnts[b]
            for j in range(4 * W // 16):
                sbufs[b][pl.ds(cb + j * 16, 16)] = pad_src
                dbufs[b][pl.ds(cb + j * 16, 16)] = pad_dst
            padded = ((cb + (4 * W - 1)) // (4 * W)) * (4 * W)
            cw = jnp.where(lanes == b, padded, cw)
            boff = ((rel * NB + b) * NT + p) * CAP
            pltpu.sync_copy(sbufs[b], srcb.at[pl.ds(boff, CAP)])
            pltpu.sync_copy(dbufs[b], dstb.at[pl.ds(boff, CAP)])
        cvec[...] = cw
        pltpu.sync_copy(cvec, counts.at[pl.ds((rel * NT + p) * 16, 16)])


_bucketize = pl.kernel(
    _bucketize_body,
    out_type=(
        jax.ShapeDtypeStruct((L * NB * NT * CAP,), jnp.int32),
        jax.ShapeDtypeStruct((L * NB * NT * CAP,), jnp.int32),
        jax.ShapeDtypeStruct((L * NT * 16,), jnp.int32),
    ),
    mesh=_mesh,
    compiler_params=_sc_params,
    scratch_types=(
        [pltpu.VMEM((CAP,), jnp.int32)] * 8
        + [pltpu.VMEM((WIN,), jnp.int32),
           pltpu.VMEM((WIN,), jnp.int32),
           pltpu.VMEM((16,), jnp.int32)]
    ),
)


def _layer_sc_body(r_hbm, m0_hbm, m1_hbm, srcb, dstb, counts, out_hbm,
                   acc, idx0, idx1, idx2, idx3, rowsa, rowsb, cvec,
                   isem0, isem1, isem2, isem3, gsema, gsemb):
    idxs = (idx0, idx1, idx2, idx3)
    isems = (isem0, isem1, isem2, isem3)
    c = lax.axis_index("c")
    s = lax.axis_index("s")
    lanes = jnp.arange(16, dtype=jnp.int32)
    for cpass in range(2):
        b = c * 2 + cpass
        pltpu.sync_copy(r_hbm.at[pl.ds(b * BUCKET + s * RPT, RPT)],
                        acc.at[pl.ds(s * RPT, RPT)])
        plsc.subcore_barrier()
        for rel in range(L):
            m_hbm = (m0_hbm, m1_hbm)[rel]
            for prod_off in (0, 16):
                prod = s + prod_off
                pltpu.sync_copy(counts.at[pl.ds((rel * NT + prod) * 16, 16)],
                                cvec)
                cnt = jnp.max(jnp.where(lanes == b, cvec[...], 0))
                base = ((rel * NB + b) * NT + prod) * CAP
                nwin = cnt // W

                def start_idx(w, j):
                    pltpu.async_copy(
                        srcb.at[pl.ds(base + w * W, W)], idxs[j].at[0],
                        isems[j])
                    pltpu.async_copy(
                        dstb.at[pl.ds(base + w * W, W)], idxs[j].at[1],
                        isems[j])

                def wait_idx(w, j):
                    pltpu.make_async_copy(
                        srcb.at[pl.ds(base + w * W, W)], idxs[j].at[0],
                        isems[j]).wait()
                    pltpu.make_async_copy(
                        dstb.at[pl.ds(base + w * W, W)], idxs[j].at[1],
                        isems[j]).wait()

                def gstart(j, rows, gsem):
                    pltpu.async_copy(m_hbm.at[idxs[j].at[0]], rows, gsem)

                def gwait(j, rows, gsem):
                    pltpu.make_async_copy(
                        m_hbm.at[idxs[j].at[0]], rows, gsem).wait()

                def scat(j, rows):
                    pltpu.sync_copy(rows, acc.at[idxs[j].at[1]], add=True)

                @pl.when(nwin > 0)
                def _():
                    start_idx(0, 0)
                    start_idx(1, 1)
                    wait_idx(0, 0)
                    gstart(0, rowsa, gsema)

                    def body4(w4, carry):
                        w = 4 * w4
                        wait_idx(w + 1, 1)
                        gstart(1, rowsb, gsemb)
                        start_idx(w + 2, 2)
                        start_idx(w + 3, 3)
                        gwait(0, rowsa, gsema)
                        scat(0, rowsa)
                        wait_idx(w + 2, 2)
                        gstart(2, rowsa, gsema)
                        gwait(1, rowsb, gsemb)
                        scat(1, rowsb)
                        wait_idx(w + 3, 3)
                        gstart(3, rowsb, gsemb)

                        @pl.when(w + 4 < nwin)
                        def _():
                            start_idx(w + 4, 0)
                            start_idx(w + 5, 1)

                        gwait(2, rowsa, gsema)
                        scat(2, rowsa)

                        @pl.when(w + 4 < nwin)
                        def _():
                            wait_idx(w + 4, 0)
                            gstart(0, rowsa, gsema)

                        gwait(3, rowsb, gsemb)
                        scat(3, rowsb)
                        return carry

                    lax.fori_loop(0, nwin // 4, body4, 0)
        plsc.subcore_barrier()
        pltpu.sync_copy(acc.at[pl.ds(s * RPT, RPT)],
                        out_hbm.at[pl.ds(b * BUCKET + s * RPT, RPT)])
        plsc.subcore_barrier()


_layer_sc = pl.kernel(
    _layer_sc_body,
    out_type=jax.ShapeDtypeStruct((NP, D), jnp.float32),
    mesh=_mesh,
    compiler_params=_sc_params,
    scratch_types=[
        pltpu.VMEM_SHARED((ACC_ROWS, D), jnp.float32),
        pltpu.VMEM((2, W), jnp.int32),
        pltpu.VMEM((2, W), jnp.int32),
        pltpu.VMEM((2, W), jnp.int32),
        pltpu.VMEM((2, W), jnp.int32),
        pltpu.VMEM((W, D), jnp.float32),
        pltpu.VMEM((W, D), jnp.float32),
        pltpu.VMEM((16,), jnp.int32),
        pltpu.SemaphoreType.DMA,
        pltpu.SemaphoreType.DMA,
        pltpu.SemaphoreType.DMA,
        pltpu.SemaphoreType.DMA,
        pltpu.SemaphoreType.DMA,
        pltpu.SemaphoreType.DMA,
    ],
)


def _mats_body(do_relu, h_ref, wrt_ref, br_ref, wc0t_ref, wc1t_ref,
               wet_ref, be_ref, r_ref, m0_ref, m1_ref):
    h = h_ref[...]
    if do_relu:
        h = jnp.maximum(h, 0.0)
    if wet_ref is not None:
        h = (jnp.dot(h, wet_ref[...], preferred_element_type=jnp.float32)
             + be_ref[...])
    r_ref[...] = (jnp.dot(h, wrt_ref[...], preferred_element_type=jnp.float32)
                  + br_ref[...])
    m0_ref[...] = jnp.dot(h, wc0t_ref[...], preferred_element_type=jnp.float32)
    m1_ref[...] = jnp.dot(h, wc1t_ref[...], preferred_element_type=jnp.float32)


def _make_mats(with_emb, do_relu):
    full = pl.BlockSpec((D, D), lambda i: (0, 0))
    bias = pl.BlockSpec((1, D), lambda i: (0, 0))
    row = pl.BlockSpec((R_BLK, D), lambda i: (i, 0))
    in_specs = [row, full, bias, full, full]
    if with_emb:
        in_specs += [full, bias]

    def body(*refs):
        if with_emb:
            _mats_body(do_relu, *refs[:7], *refs[7:])
        else:
            _mats_body(do_relu, *refs[:5], None, None, *refs[5:])

    return pl.pallas_call(
        body,
        grid=(N_BLKS,),
        in_specs=in_specs,
        out_specs=[row, row, row],
        out_shape=[jax.ShapeDtypeStruct((NP, D), jnp.float32)] * 3,
    )


_mats_emb = _make_mats(True, False)
_mats_relu = _make_mats(False, True)


def _pool_body(h_ref, batch_ref, w1t_ref, b1_ref, w2t_ref, b2_ref,
               out_ref, acc_ref):
    i = pl.program_id(0)

    @pl.when(i == 0)
    def _():
        acc_ref[...] = jnp.zeros_like(acc_ref)

    bvec = batch_ref[0, 0, :]
    onehot = (bvec[:, None]
              == lax.broadcasted_iota(jnp.int32, (R_BLK, G), 1)
              ).astype(jnp.float32)
    acc_ref[...] += lax.dot_general(
        onehot, h_ref[...], (((0,), (0,)), ((), ())),
        preferred_element_type=jnp.float32)

    @pl.when(i == N_BLKS - 1)
    def _():
        hid = jnp.maximum(
            jnp.dot(acc_ref[...], w1t_ref[...],
                    preferred_element_type=jnp.float32) + b1_ref[...], 0.0)
        o = jnp.dot(hid, w2t_ref[...],
                    preferred_element_type=jnp.float32) + b2_ref[...]
        out_ref[...] = o.reshape(1, G)


_pool = pl.pallas_call(
    _pool_body,
    grid=(N_BLKS,),
    in_specs=[
        pl.BlockSpec((R_BLK, D), lambda i: (i, 0)),
        pl.BlockSpec((1, 1, R_BLK), lambda i: (i, 0, 0)),
        pl.BlockSpec((D, D), lambda i: (0, 0)),
        pl.BlockSpec((1, D), lambda i: (0, 0)),
        pl.BlockSpec((D, 1), lambda i: (0, 0)),
        pl.BlockSpec((1, 1), lambda i: (0, 0)),
    ],
    out_specs=pl.BlockSpec((1, G), lambda i: (0, 0)),
    out_shape=jax.ShapeDtypeStruct((1, G), jnp.float32),
    scratch_shapes=[pltpu.VMEM((G, D), jnp.float32)],
)


def kernel(x, edge_index, batch, W_emb, b_emb, W_root, b_root, W_conv,
           W1, b1, W2, b2):
    src = edge_index[:, 0, :]
    dst = edge_index[:, 1, :]
    pad = EPAD - E
    src_flat = jnp.concatenate(
        [src, jnp.zeros((L, pad), jnp.int32)], axis=1).reshape(-1)
    dst_flat = jnp.concatenate(
        [dst, jnp.full((L, pad), -1, jnp.int32)], axis=1).reshape(-1)
    srcb, dstb, counts = _bucketize(src_flat, dst_flat)

    h = jnp.pad(x, ((0, NP - N), (0, 0)))
    for l in range(NLAYERS):
        mats = _mats_emb if l == 0 else _mats_relu
        args = [h, W_root[l].T, b_root[l].reshape(1, D),
                W_conv[l, 0].T, W_conv[l, 1].T]
        if l == 0:
            args += [W_emb.T, b_emb.reshape(1, D)]
        r, m0, m1 = mats(*args)
        h = _layer_sc(r, m0, m1, srcb, dstb, counts)

    batch_p = jnp.pad(batch, (0, NP - N), constant_values=-1)
    batch3d = batch_p.reshape(N_BLKS, 1, R_BLK)
    out = _pool(h, batch3d, W1.T, b1.reshape(1, D), W2.T, b2.reshape(1, 1))
    return out.reshape(G)


# speedup vs baseline: 6.7007x; 1.0192x over previous
import jax
import jax.numpy as jnp
from jax import lax
from jax.experimental import pallas as pl
from jax.experimental.pallas import tpu as pltpu
from jax.experimental.pallas import tpu_sc as plsc

N, D, L, G, E, NLAYERS = 50000, 128, 2, 64, 400000, 4

NB = 4
BUCKET = 12544
NP = NB * BUCKET
ACC_ROWS = 12548
NT = 32
SLICE = 12544
EPAD = NT * SLICE
W = 104
CAP = SLICE + 8 * W
NWIN = 8
WIN = SLICE // NWIN
RPT = BUCKET // 16

R_BLK = 784
N_BLKS = NP // R_BLK

_mesh = plsc.VectorSubcoreMesh(core_axis_name="c", subcore_axis_name="s")
_sc_params = pltpu.CompilerParams(needs_layout_passes=False)


def _bucketize_body(src2d, dst2d, srcb, dstb, counts, sb0, sb1, sb2, sb3,
                    db0, db1, db2, db3, swin, dwin, cvec):
    sbufs = (sb0, sb1, sb2, sb3)
    dbufs = (db0, db1, db2, db3)
    c = lax.axis_index("c")
    s = lax.axis_index("s")
    p = c * 16 + s
    lanes = jnp.arange(16, dtype=jnp.int32)
    pad_src = lanes
    pad_dst = BUCKET + (lanes & 3)
    for rel in range(L):
        cnts = (jnp.int32(0),) * NB
        for w in range(NWIN):
            off = rel * EPAD + p * SLICE + w * WIN
            pltpu.sync_copy(src2d.at[pl.ds(off, WIN)], swin)
            pltpu.sync_copy(dst2d.at[pl.ds(off, WIN)], dwin)

            def body(k, cnts):
                sv = swin[pl.ds(k * 16, 16)]
                dv = dwin[pl.ds(k * 16, 16)]
                bkt = ((dv >= BUCKET).astype(jnp.int32)
                       + (dv >= 2 * BUCKET).astype(jnp.int32)
                       + (dv >= 3 * BUCKET).astype(jnp.int32))
                valid = dv >= 0
                dloc = dv - bkt * BUCKET
                out = []
                for b in range(NB):
                    msk = jnp.logical_and(bkt == b, valid)
                    plsc.store_compressed(
                        sbufs[b].at[pl.ds(cnts[b], 16)], sv, mask=msk)
                    plsc.store_compressed(
                        dbufs[b].at[pl.ds(cnts[b], 16)], dloc, mask=msk)
                    npop = jnp.max(plsc.all_reduce_population_count(msk))
                    out.append(cnts[b] + npop)
                return tuple(out)

            cnts = lax.fori_loop(0, WIN // 16, body, cnts)
        cw = jnp.zeros((16,), jnp.int32)
        for b in range(NB):
            cb = cnts[b]
            for j in range(8 * W // 16):
                sbufs[b][pl.ds(cb + j * 16, 16)] = pad_src
                dbufs[b][pl.ds(cb + j * 16, 16)] = pad_dst
            padded = ((cb + (8 * W - 1)) // (8 * W)) * (8 * W)
            cw = jnp.where(lanes == b, padded, cw)
            boff = ((rel * NB + b) * NT + p) * CAP
            pltpu.sync_copy(sbufs[b], srcb.at[pl.ds(boff, CAP)])
            pltpu.sync_copy(dbufs[b], dstb.at[pl.ds(boff, CAP)])
        cvec[...] = cw
        pltpu.sync_copy(cvec, counts.at[pl.ds((rel * NT + p) * 16, 16)])


_bucketize = pl.kernel(
    _bucketize_body,
    out_type=(
        jax.ShapeDtypeStruct((L * NB * NT * CAP,), jnp.int32),
        jax.ShapeDtypeStruct((L * NB * NT * CAP,), jnp.int32),
        jax.ShapeDtypeStruct((L * NT * 16,), jnp.int32),
    ),
    mesh=_mesh,
    compiler_params=_sc_params,
    scratch_types=(
        [pltpu.VMEM((CAP,), jnp.int32)] * 8
        + [pltpu.VMEM((WIN,), jnp.int32),
           pltpu.VMEM((WIN,), jnp.int32),
           pltpu.VMEM((16,), jnp.int32)]
    ),
)


def _layer_sc_body(r_hbm, m0_hbm, m1_hbm, srcb, dstb, counts, out_hbm,
                   acc, sblka, sblkb, dblka, dblkb, rowsa, rowsb, cvec,
                   bsema, bsemb, gsema, gsemb):
    c = lax.axis_index("c")
    s = lax.axis_index("s")
    lanes = jnp.arange(16, dtype=jnp.int32)
    BW = 4 * W
    for cpass in range(2):
        b = c * 2 + cpass
        pltpu.sync_copy(r_hbm.at[pl.ds(b * BUCKET + s * RPT, RPT)],
                        acc.at[pl.ds(s * RPT, RPT)])
        plsc.subcore_barrier()
        for rel in range(L):
            m_hbm = (m0_hbm, m1_hbm)[rel]
            for prod_off in (0, 16):
                prod = s + prod_off
                pltpu.sync_copy(counts.at[pl.ds((rel * NT + prod) * 16, 16)],
                                cvec)
                cnt = jnp.max(jnp.where(lanes == b, cvec[...], 0))
                base = ((rel * NB + b) * NT + prod) * CAP
                nblk = cnt // BW

                def fill_blk(k, sblk, dblk, bsem):
                    off = base + k * BW
                    pltpu.async_copy(srcb.at[pl.ds(off, BW)], sblk, bsem)
                    for j in range(4):
                        pltpu.async_copy(dstb.at[pl.ds(off + j * W, W)],
                                         dblk.at[j], bsem)

                def wait_blk(k, sblk, dblk, bsem):
                    off = base + k * BW
                    pltpu.make_async_copy(
                        srcb.at[pl.ds(off, BW)], sblk, bsem).wait()
                    for j in range(4):
                        pltpu.make_async_copy(
                            dstb.at[pl.ds(off + j * W, W)], dblk.at[j],
                            bsem).wait()

                def gstart(sblk, j, rows, gsem):
                    pltpu.async_copy(
                        m_hbm.at[sblk.at[pl.ds(j * W, W)]], rows, gsem)

                def gwait(sblk, j, rows, gsem):
                    pltpu.make_async_copy(
                        m_hbm.at[sblk.at[pl.ds(j * W, W)]], rows, gsem).wait()

                def scat(dblk, j, rows):
                    pltpu.sync_copy(rows, acc.at[dblk.at[j]], add=True)

                @pl.when(nblk > 0)
                def _():
                    fill_blk(0, sblka, dblka, bsema)
                    fill_blk(1, sblkb, dblkb, bsemb)
                    wait_blk(0, sblka, dblka, bsema)
                    gstart(sblka, 0, rowsa, gsema)

                    def bodyp(i, carry):
                        ka = 2 * i
                        gstart(sblka, 1, rowsb, gsemb)
                        gwait(sblka, 0, rowsa, gsema)
                        scat(dblka, 0, rowsa)
                        gstart(sblka, 2, rowsa, gsema)
                        gwait(sblka, 1, rowsb, gsemb)
                        scat(dblka, 1, rowsb)
                        gstart(sblka, 3, rowsb, gsemb)
                        wait_blk(ka + 1, sblkb, dblkb, bsemb)
                        gwait(sblka, 2, rowsa, gsema)
                        scat(dblka, 2, rowsa)
                        gstart(sblkb, 0, rowsa, gsema)
                        gwait(sblka, 3, rowsb, gsemb)
                        scat(dblka, 3, rowsb)

                        @pl.when(ka + 2 < nblk)
                        def _():
                            fill_blk(ka + 2, sblka, dblka, bsema)

                        gstart(sblkb, 1, rowsb, gsemb)
                        gwait(sblkb, 0, rowsa, gsema)
                        scat(dblkb, 0, rowsa)
                        gstart(sblkb, 2, rowsa, gsema)
                        gwait(sblkb, 1, rowsb, gsemb)
                        scat(dblkb, 1, rowsb)
                        gstart(sblkb, 3, rowsb, gsemb)
                        gwait(sblkb, 2, rowsa, gsema)
                        scat(dblkb, 2, rowsa)

                        @pl.when(ka + 2 < nblk)
                        def _():
                            wait_blk(ka + 2, sblka, dblka, bsema)
                            gstart(sblka, 0, rowsa, gsema)
                            fill_blk(ka + 3, sblkb, dblkb, bsemb)

                        gwait(sblkb, 3, rowsb, gsemb)
                        scat(dblkb, 3, rowsb)
                        return carry

                    lax.fori_loop(0, nblk // 2, bodyp, 0)
        plsc.subcore_barrier()
        pltpu.sync_copy(acc.at[pl.ds(s * RPT, RPT)],
                        out_hbm.at[pl.ds(b * BUCKET + s * RPT, RPT)])
        plsc.subcore_barrier()


_layer_sc = pl.kernel(
    _layer_sc_body,
    out_type=jax.ShapeDtypeStruct((NP, D), jnp.float32),
    mesh=_mesh,
    compiler_params=_sc_params,
    scratch_types=[
        pltpu.VMEM_SHARED((ACC_ROWS, D), jnp.float32),
        pltpu.VMEM((4 * W,), jnp.int32),
        pltpu.VMEM((4 * W,), jnp.int32),
        pltpu.VMEM((4, W), jnp.int32),
        pltpu.VMEM((4, W), jnp.int32),
        pltpu.VMEM((W, D), jnp.float32),
        pltpu.VMEM((W, D), jnp.float32),
        pltpu.VMEM((16,), jnp.int32),
        pltpu.SemaphoreType.DMA,
        pltpu.SemaphoreType.DMA,
        pltpu.SemaphoreType.DMA,
        pltpu.SemaphoreType.DMA,
    ],
)


def _mats_body(do_relu, h_ref, wrt_ref, br_ref, wc0t_ref, wc1t_ref,
               wet_ref, be_ref, r_ref, m0_ref, m1_ref):
    h = h_ref[...]
    if do_relu:
        h = jnp.maximum(h, 0.0)
    if wet_ref is not None:
        h = (jnp.dot(h, wet_ref[...], preferred_element_type=jnp.float32)
             + be_ref[...])
    r_ref[...] = (jnp.dot(h, wrt_ref[...], preferred_element_type=jnp.float32)
                  + br_ref[...])
    m0_ref[...] = jnp.dot(h, wc0t_ref[...], preferred_element_type=jnp.float32)
    m1_ref[...] = jnp.dot(h, wc1t_ref[...], preferred_element_type=jnp.float32)


def _make_mats(with_emb, do_relu):
    full = pl.BlockSpec((D, D), lambda i: (0, 0))
    bias = pl.BlockSpec((1, D), lambda i: (0, 0))
    row = pl.BlockSpec((R_BLK, D), lambda i: (i, 0))
    in_specs = [row, full, bias, full, full]
    if with_emb:
        in_specs += [full, bias]

    def body(*refs):
        if with_emb:
            _mats_body(do_relu, *refs[:7], *refs[7:])
        else:
            _mats_body(do_relu, *refs[:5], None, None, *refs[5:])

    return pl.pallas_call(
        body,
        grid=(N_BLKS,),
        in_specs=in_specs,
        out_specs=[row, row, row],
        out_shape=[jax.ShapeDtypeStruct((NP, D), jnp.float32)] * 3,
    )


_mats_emb = _make_mats(True, False)
_mats_relu = _make_mats(False, True)


def _pool_body(h_ref, batch_ref, w1t_ref, b1_ref, w2t_ref, b2_ref,
               out_ref, acc_ref):
    i = pl.program_id(0)

    @pl.when(i == 0)
    def _():
        acc_ref[...] = jnp.zeros_like(acc_ref)

    bvec = batch_ref[0, 0, :]
    onehot = (bvec[:, None]
              == lax.broadcasted_iota(jnp.int32, (R_BLK, G), 1)
              ).astype(jnp.float32)
    acc_ref[...] += lax.dot_general(
        onehot, h_ref[...], (((0,), (0,)), ((), ())),
        preferred_element_type=jnp.float32)

    @pl.when(i == N_BLKS - 1)
    def _():
        hid = jnp.maximum(
            jnp.dot(acc_ref[...], w1t_ref[...],
                    preferred_element_type=jnp.float32) + b1_ref[...], 0.0)
        o = jnp.dot(hid, w2t_ref[...],
                    preferred_element_type=jnp.float32) + b2_ref[...]
        out_ref[...] = o.reshape(1, G)


_pool = pl.pallas_call(
    _pool_body,
    grid=(N_BLKS,),
    in_specs=[
        pl.BlockSpec((R_BLK, D), lambda i: (i, 0)),
        pl.BlockSpec((1, 1, R_BLK), lambda i: (i, 0, 0)),
        pl.BlockSpec((D, D), lambda i: (0, 0)),
        pl.BlockSpec((1, D), lambda i: (0, 0)),
        pl.BlockSpec((D, 1), lambda i: (0, 0)),
        pl.BlockSpec((1, 1), lambda i: (0, 0)),
    ],
    out_specs=pl.BlockSpec((1, G), lambda i: (0, 0)),
    out_shape=jax.ShapeDtypeStruct((1, G), jnp.float32),
    scratch_shapes=[pltpu.VMEM((G, D), jnp.float32)],
)


def kernel(x, edge_index, batch, W_emb, b_emb, W_root, b_root, W_conv,
           W1, b1, W2, b2):
    src = edge_index[:, 0, :]
    dst = edge_index[:, 1, :]
    pad = EPAD - E
    src_flat = jnp.concatenate(
        [src, jnp.zeros((L, pad), jnp.int32)], axis=1).reshape(-1)
    dst_flat = jnp.concatenate(
        [dst, jnp.full((L, pad), -1, jnp.int32)], axis=1).reshape(-1)
    srcb, dstb, counts = _bucketize(src_flat, dst_flat)

    h = jnp.pad(x, ((0, NP - N), (0, 0)))
    for l in range(NLAYERS):
        mats = _mats_emb if l == 0 else _mats_relu
        args = [h, W_root[l].T, b_root[l].reshape(1, D),
                W_conv[l, 0].T, W_conv[l, 1].T]
        if l == 0:
            args += [W_emb.T, b_emb.reshape(1, D)]
        r, m0, m1 = mats(*args)
        h = _layer_sc(r, m0, m1, srcb, dstb, counts)

    batch_p = jnp.pad(batch, (0, NP - N), constant_values=-1)
    batch3d = batch_p.reshape(N_BLKS, 1, R_BLK)
    out = _pool(h, batch3d, W1.T, b1.reshape(1, D), W2.T, b2.reshape(1, 1))
    return out.reshape(G)
